# baseline (speedup 1.0000x reference)
import functools

import jax
import jax.numpy as jnp
from jax import lax
from jax.experimental import pallas as pl
from jax.experimental.pallas import tpu as pltpu
from jax.experimental.pallas import tpu_sc as plsc

N = 10000
E = 320000
D = 128
NC = 2
NS = 16
NW = NC * NS
K = 128
RND = 80
EPW = RND * K
NPAD = 10240
SPT = NPAD // NS
BLK = 1000
CW = 16


def _seg_body(nph, with_count, async_scatter, *refs):
    it = iter(refs)
    P_list = [next(it) for _ in range(nph)]
    src_hbm = next(it)
    dst_hbm = next(it)
    ones_hbm = next(it)
    zrow_hbm = next(it)
    zcnt_hbm = next(it)
    S_list = [next(it) for _ in range(nph)]
    C_hbm = next(it) if with_count else None
    src_v = next(it)
    dst_v = next(it)
    buf0 = next(it)
    buf1 = next(it)
    ones_v = next(it) if with_count else None
    acc = next(it)
    cacc = next(it) if with_count else None
    gsem0 = next(it)
    gsem1 = next(it)
    ssem0 = next(it) if async_scatter else None
    ssem1 = next(it) if async_scatter else None

    c = lax.axis_index("c")
    s = lax.axis_index("s")
    wid = c * NS + s
    stripe = pl.ds(s * SPT, SPT)

    pltpu.sync_copy(src_hbm.at[wid], src_v)
    pltpu.sync_copy(dst_hbm.at[wid], dst_v)
    if with_count:
        pltpu.sync_copy(ones_hbm, ones_v)

    for ph in range(nph):
        count_now = with_count and ph == 0
        P_hbm = P_list[ph]

        pltpu.sync_copy(zrow_hbm, acc.at[stripe])
        if count_now:
            pltpu.sync_copy(zcnt_hbm, cacc.at[stripe])
        plsc.subcore_barrier()

        def start_gather(j, buf, sem):
            pltpu.async_copy(P_hbm.at[src_v.at[j]], buf, sem)

        def finish(j, buf, sem):
            pltpu.make_async_copy(P_hbm.at[src_v.at[j]], buf, sem).wait()
            pltpu.sync_copy(buf, acc.at[dst_v.at[j]], add=True)
            if count_now:
                pltpu.sync_copy(ones_v, cacc.at[dst_v.at[j]], add=True)

        if not async_scatter:
            start_gather(0, buf0, gsem0)

            @pl.loop(0, RND - 2, step=2)
            def _(j):
                start_gather(j + 1, buf1, gsem1)
                finish(j, buf0, gsem0)
                start_gather(j + 2, buf0, gsem0)
                finish(j + 1, buf1, gsem1)

            start_gather(RND - 1, buf1, gsem1)
            finish(RND - 2, buf0, gsem0)
            finish(RND - 1, buf1, gsem1)
        else:
            def absorb(j, buf, gsem, ssem):
                pltpu.make_async_copy(P_hbm.at[src_v.at[j]], buf, gsem).wait()
                return pltpu.async_copy(buf, acc.at[dst_v.at[j]], ssem,
                                        add=True)

            start_gather(0, buf0, gsem0)
            start_gather(1, buf1, gsem1)

            @pl.loop(0, RND - 2, step=2)
            def _(j):
                d0 = absorb(j, buf0, gsem0, ssem0)
                d1 = absorb(j + 1, buf1, gsem1, ssem1)
                d0.wait()
                start_gather(j + 2, buf0, gsem0)
                d1.wait()
                start_gather(j + 3, buf1, gsem1)

            d0 = absorb(RND - 2, buf0, gsem0, ssem0)
            d1 = absorb(RND - 1, buf1, gsem1, ssem1)
            d0.wait()
            d1.wait()

        plsc.subcore_barrier()
        pltpu.sync_copy(acc.at[stripe], S_list[ph].at[c, stripe])
        if count_now:
            pltpu.sync_copy(cacc.at[stripe], C_hbm.at[c, stripe])


def _make_segsum(nph, W, dtype, with_count, async_scatter=False):
    mesh = plsc.VectorSubcoreMesh(core_axis_name="c", subcore_axis_name="s",
                                  num_cores=NC, num_subcores=NS)
    outs = [jax.ShapeDtypeStruct((NC, NPAD, W), dtype) for _ in range(nph)]
    if with_count:
        outs.append(jax.ShapeDtypeStruct((NC, NPAD, CW), jnp.float32))
    scratch = [
        pltpu.VMEM((RND, K), jnp.int32),
        pltpu.VMEM((RND, K), jnp.int32),
        pltpu.VMEM((K, W), dtype),
        pltpu.VMEM((K, W), dtype),
    ]
    if with_count:
        scratch.append(pltpu.VMEM((K, CW), jnp.float32))
    scratch.append(pltpu.VMEM_SHARED((NPAD, W), dtype))
    if with_count:
        scratch.append(pltpu.VMEM_SHARED((NPAD, CW), jnp.float32))
    scratch += [pltpu.SemaphoreType.DMA] * (4 if async_scatter else 2)
    return pl.kernel(
        functools.partial(_seg_body, nph, with_count, async_scatter),
        out_type=outs,
        mesh=mesh,
        scratch_types=scratch,
        compiler_params=pltpu.CompilerParams(use_tc_tiling_on_sc=False),
    )


@functools.cache
def _segsum(nph, W, dtype, with_count, async_scatter=False):
    return _make_segsum(nph, W, dtype, with_count, async_scatter)


def _tc_matmul(x, WT, widths, dtypes):
    nrow = x.shape[0]
    din = x.shape[1]
    wn2 = WT.shape[1]
    assert sum(widths) == wn2
    offs = [sum(widths[:i]) for i in range(len(widths))]

    def body(x_ref, w_ref, *out_refs):
        pr = jnp.dot(x_ref[...], w_ref[...], preferred_element_type=jnp.float32)
        for o_ref, off, w, dt in zip(out_refs, offs, widths, dtypes):
            o_ref[...] = pr[:, off:off + w].astype(dt)

    return pl.pallas_call(
        body,
        grid=(nrow // BLK,),
        in_specs=[pl.BlockSpec((BLK, din), lambda i: (i, 0)),
                  pl.BlockSpec((din, wn2), lambda i: (0, 0))],
        out_specs=[pl.BlockSpec((BLK, w), lambda i: (i, 0)) for w in widths],
        out_shape=[jax.ShapeDtypeStruct((nrow, w), dt)
                   for w, dt in zip(widths, dtypes)],
    )(x, WT)


def _tc_mid(Sp, Cp, R, b, scale, shift, WT, widths, dtypes):
    wn2 = WT.shape[1]
    offs = [sum(widths[:i]) for i in range(len(widths))]

    def body(s_ref, c_ref, r_ref, b_ref, sc_ref, sh_ref, w_ref, *out_refs):
        ssum = (s_ref[0].astype(jnp.float32) + s_ref[1].astype(jnp.float32))
        cnt = c_ref[0, :, 0:1] + c_ref[1, :, 0:1]
        h = ssum / jnp.maximum(cnt, 1.0) + b_ref[...] + r_ref[...]
        h = jnp.maximum(h * sc_ref[...] + sh_ref[...], 0.0)
        pr = jnp.dot(h, w_ref[...], preferred_element_type=jnp.float32)
        for o_ref, off, w, dt in zip(out_refs, offs, widths, dtypes):
            o_ref[...] = pr[:, off:off + w].astype(dt)

    return pl.pallas_call(
        body,
        grid=(N // BLK,),
        in_specs=[pl.BlockSpec((NC, BLK, D), lambda i: (0, i, 0)),
                  pl.BlockSpec((NC, BLK, CW), lambda i: (0, i, 0)),
                  pl.BlockSpec((BLK, D), lambda i: (i, 0)),
                  pl.BlockSpec((1, D), lambda i: (0, 0)),
                  pl.BlockSpec((1, D), lambda i: (0, 0)),
                  pl.BlockSpec((1, D), lambda i: (0, 0)),
                  pl.BlockSpec((D, wn2), lambda i: (0, 0))],
        out_specs=[pl.BlockSpec((BLK, w), lambda i: (i, 0)) for w in widths],
        out_shape=[jax.ShapeDtypeStruct((N, w), dt)
                   for w, dt in zip(widths, dtypes)],
    )(Sp, Cp, R, b, scale, shift, WT)


def _tc_fin(Sp, Cp, R, b, C_out):
    def body(s_ref, c_ref, r_ref, b_ref, o_ref):
        ssum = s_ref[0] + s_ref[1]
        cnt = c_ref[0, :, 0:1] + c_ref[1, :, 0:1]
        o = ssum / jnp.maximum(cnt, 1.0) + b_ref[...] + r_ref[...]
        o_ref[...] = o[:, :C_out]

    return pl.pallas_call(
        body,
        grid=(N // BLK,),
        in_specs=[pl.BlockSpec((NC, BLK, CW), lambda i: (0, i, 0)),
                  pl.BlockSpec((NC, BLK, CW), lambda i: (0, i, 0)),
                  pl.BlockSpec((BLK, CW), lambda i: (i, 0)),
                  pl.BlockSpec((1, CW), lambda i: (0, 0))],
        out_specs=pl.BlockSpec((BLK, C_out), lambda i: (i, 0)),
        out_shape=jax.ShapeDtypeStruct((N, C_out), jnp.float32),
    )(Sp, Cp, R, b)


def kernel(x, edge_index, Wl0, Wr0, b0, g0, be0, rm0, rv0,
           Wl1, Wr1, b1, g1, be1, rm1, rv1, Wl2, Wr2, b2):
    C_out = Wl2.shape[0]

    WT0 = jnp.concatenate([Wl0.T, Wr0.T], axis=1)
    WT1 = jnp.concatenate([Wl1.T, Wr1.T], axis=1)
    pad = ((0, CW - C_out), (0, 0))
    WT2 = jnp.concatenate([jnp.pad(Wl2, pad).T, jnp.pad(Wr2, pad).T], axis=1)

    scale0 = (g0 / jnp.sqrt(rv0 + 1e-5)).reshape(1, D)
    shift0 = (be0 - rm0 * scale0[0]).reshape(1, D)
    scale1 = (g1 / jnp.sqrt(rv1 + 1e-5)).reshape(1, D)
    shift1 = (be1 - rm1 * scale1[0]).reshape(1, D)
    b0_ = b0.reshape(1, D)
    b1_ = b1.reshape(1, D)
    b2_ = jnp.pad(b2, (0, CW - C_out)).reshape(1, CW)

    npad_edges = NW * EPW - E
    iota_pad = jnp.arange(npad_edges, dtype=jnp.int32)
    src_p = jnp.concatenate(
        [edge_index[0], jax.lax.rem(iota_pad * 37, jnp.int32(N))])
    dst_p = jnp.concatenate(
        [edge_index[1],
         N + jax.lax.rem(iota_pad * 37, jnp.int32(NPAD - N))])
    src3 = src_p.reshape(NW, RND, K)
    dst3 = dst_p.reshape(NW, RND, K)
    ones16 = jnp.ones((K, CW), jnp.float32)
    zbf = jnp.zeros((SPT, D), jnp.bfloat16)
    z16 = jnp.zeros((SPT, CW), jnp.float32)
    f32, bf16 = jnp.float32, jnp.bfloat16

    P0, R0 = _tc_matmul(x, WT0, (D, D), (bf16, f32))
    S0, CNT = jax.tree.leaves(
        _segsum(1, D, bf16, True)(P0, src3, dst3, ones16, zbf, z16))
    P1, R1 = _tc_mid(S0, CNT, R0, b0_, scale0, shift0,
                     WT1, (D, D), (bf16, f32))
    (S1,) = jax.tree.leaves(
        _segsum(1, D, bf16, False)(P1, src3, dst3, ones16, zbf, z16))
    P2, R2 = _tc_mid(S1, CNT, R1, b1_, scale1, shift1,
                     WT2, (CW, CW), (f32, f32))
    (S2,) = jax.tree.leaves(
        _segsum(1, CW, f32, False, True)(P2, src3, dst3, ones16, z16, z16))
    return _tc_fin(S2, CNT, R2, b2_, C_out)


# speedup vs baseline: 1.0069x; 1.0069x over previous
import functools

import jax
import jax.numpy as jnp
from jax import lax
from jax.experimental import pallas as pl
from jax.experimental.pallas import tpu as pltpu
from jax.experimental.pallas import tpu_sc as plsc

N = 10000
E = 320000
D = 128
NC = 2
NS = 16
NW = NC * NS
K = 128
RND = 80
EPW = RND * K
NPAD = 10240
SPT = NPAD // NS
BLK = 1000
CW = 16


def _seg_body(nph, with_count, async_scatter, *refs):
    it = iter(refs)
    P_list = [next(it) for _ in range(nph)]
    src_hbm = next(it)
    dst_hbm = next(it)
    ones_hbm = next(it)
    zrow_hbm = next(it)
    zcnt_hbm = next(it)
    S_list = [next(it) for _ in range(nph)]
    C_hbm = next(it) if with_count else None
    src_v = next(it)
    dst_v = next(it)
    buf0 = next(it)
    buf1 = next(it)
    ones_v = next(it) if with_count else None
    acc = next(it)
    cacc = next(it) if with_count else None
    gsem0 = next(it)
    gsem1 = next(it)
    ssem0 = next(it) if async_scatter else None
    ssem1 = next(it) if async_scatter else None

    c = lax.axis_index("c")
    s = lax.axis_index("s")
    wid = c * NS + s
    stripe = pl.ds(s * SPT, SPT)

    pltpu.sync_copy(src_hbm.at[wid], src_v)
    pltpu.sync_copy(dst_hbm.at[wid], dst_v)
    if with_count:
        pltpu.sync_copy(ones_hbm, ones_v)

    for ph in range(nph):
        count_now = with_count and ph == 0
        P_hbm = P_list[ph]

        pltpu.sync_copy(zrow_hbm, acc.at[stripe])
        if count_now:
            pltpu.sync_copy(zcnt_hbm, cacc.at[stripe])
        plsc.subcore_barrier()

        def start_gather(j, buf, sem):
            pltpu.async_copy(P_hbm.at[src_v.at[j]], buf, sem)

        def finish(j, buf, sem):
            pltpu.make_async_copy(P_hbm.at[src_v.at[j]], buf, sem).wait()
            pltpu.sync_copy(buf, acc.at[dst_v.at[j]], add=True)
            if count_now:
                pltpu.sync_copy(ones_v, cacc.at[dst_v.at[j]], add=True)

        if not async_scatter:
            start_gather(0, buf0, gsem0)

            @pl.loop(0, RND - 2, step=2)
            def _(j):
                start_gather(j + 1, buf1, gsem1)
                finish(j, buf0, gsem0)
                start_gather(j + 2, buf0, gsem0)
                finish(j + 1, buf1, gsem1)

            start_gather(RND - 1, buf1, gsem1)
            finish(RND - 2, buf0, gsem0)
            finish(RND - 1, buf1, gsem1)
        else:
            def absorb(j, buf, gsem, ssem):
                pltpu.make_async_copy(P_hbm.at[src_v.at[j]], buf, gsem).wait()
                return pltpu.async_copy(buf, acc.at[dst_v.at[j]], ssem,
                                        add=True)

            start_gather(0, buf0, gsem0)
            start_gather(1, buf1, gsem1)

            @pl.loop(0, RND - 2, step=2)
            def _(j):
                d0 = absorb(j, buf0, gsem0, ssem0)
                d1 = absorb(j + 1, buf1, gsem1, ssem1)
                d0.wait()
                start_gather(j + 2, buf0, gsem0)
                d1.wait()
                start_gather(j + 3, buf1, gsem1)

            d0 = absorb(RND - 2, buf0, gsem0, ssem0)
            d1 = absorb(RND - 1, buf1, gsem1, ssem1)
            d0.wait()
            d1.wait()

        plsc.subcore_barrier()
        pltpu.sync_copy(acc.at[stripe], S_list[ph].at[c, stripe])
        if count_now:
            pltpu.sync_copy(cacc.at[stripe], C_hbm.at[c, stripe])


def _make_segsum(nph, W, dtype, with_count, async_scatter=False):
    mesh = plsc.VectorSubcoreMesh(core_axis_name="c", subcore_axis_name="s",
                                  num_cores=NC, num_subcores=NS)
    outs = [jax.ShapeDtypeStruct((NC, NPAD, W), dtype) for _ in range(nph)]
    if with_count:
        outs.append(jax.ShapeDtypeStruct((NC, NPAD, CW), jnp.float32))
    scratch = [
        pltpu.VMEM((RND, K), jnp.int32),
        pltpu.VMEM((RND, K), jnp.int32),
        pltpu.VMEM((K, W), dtype),
        pltpu.VMEM((K, W), dtype),
    ]
    if with_count:
        scratch.append(pltpu.VMEM((K, CW), jnp.float32))
    scratch.append(pltpu.VMEM_SHARED((NPAD, W), dtype))
    if with_count:
        scratch.append(pltpu.VMEM_SHARED((NPAD, CW), jnp.float32))
    scratch += [pltpu.SemaphoreType.DMA] * (4 if async_scatter else 2)
    return pl.kernel(
        functools.partial(_seg_body, nph, with_count, async_scatter),
        out_type=outs,
        mesh=mesh,
        scratch_types=scratch,
        compiler_params=pltpu.CompilerParams(use_tc_tiling_on_sc=False),
    )


@functools.cache
def _segsum(nph, W, dtype, with_count, async_scatter=False):
    return _make_segsum(nph, W, dtype, with_count, async_scatter)


def _dot_t(a, w):
    return lax.dot_general(a, w, (((1,), (1,)), ((), ())),
                           preferred_element_type=jnp.float32)


def _tc_matmul(x, Wl, Wr, dtypes):
    din = x.shape[1]
    wl_sh, wr_sh = Wl.shape, Wr.shape

    def body(x_ref, wl_ref, wr_ref, p_ref, r_ref):
        xb = x_ref[...]
        p_ref[...] = _dot_t(xb, wl_ref[...]).astype(dtypes[0])
        r_ref[...] = _dot_t(xb, wr_ref[...]).astype(dtypes[1])

    return pl.pallas_call(
        body,
        grid=(N // BLK,),
        in_specs=[pl.BlockSpec((BLK, din), lambda i: (i, 0)),
                  pl.BlockSpec(wl_sh, lambda i: (0, 0)),
                  pl.BlockSpec(wr_sh, lambda i: (0, 0))],
        out_specs=[pl.BlockSpec((BLK, wl_sh[0]), lambda i: (i, 0)),
                   pl.BlockSpec((BLK, wr_sh[0]), lambda i: (i, 0))],
        out_shape=[jax.ShapeDtypeStruct((N, wl_sh[0]), dtypes[0]),
                   jax.ShapeDtypeStruct((N, wr_sh[0]), dtypes[1])],
    )(x, Wl, Wr)


def _tc_mid(Sp, Cp, R, b, g, be, rm, rv, Wln, Wrn, dtypes):
    wl_sh, wr_sh = Wln.shape, Wrn.shape

    def body(s_ref, c_ref, r_ref, b_ref, g_ref, be_ref, rm_ref, rv_ref,
             wl_ref, wr_ref, p_ref, rn_ref):
        ssum = (s_ref[0].astype(jnp.float32) + s_ref[1].astype(jnp.float32))
        cnt = c_ref[0, :, 0:1] + c_ref[1, :, 0:1]
        scale = g_ref[...] * lax.rsqrt(rv_ref[...] + 1e-5)
        shift = be_ref[...] - rm_ref[...] * scale
        h = ssum / jnp.maximum(cnt, 1.0) + b_ref[...] + r_ref[...]
        h = jnp.maximum(h * scale + shift, 0.0)
        p_ref[...] = _dot_t(h, wl_ref[...]).astype(dtypes[0])
        rn_ref[...] = _dot_t(h, wr_ref[...]).astype(dtypes[1])

    vec = pl.BlockSpec((D,), lambda i: (0,))
    return pl.pallas_call(
        body,
        grid=(N // BLK,),
        in_specs=[pl.BlockSpec((NC, BLK, D), lambda i: (0, i, 0)),
                  pl.BlockSpec((NC, BLK, CW), lambda i: (0, i, 0)),
                  pl.BlockSpec((BLK, D), lambda i: (i, 0)),
                  vec, vec, vec, vec, vec,
                  pl.BlockSpec(wl_sh, lambda i: (0, 0)),
                  pl.BlockSpec(wr_sh, lambda i: (0, 0))],
        out_specs=[pl.BlockSpec((BLK, wl_sh[0]), lambda i: (i, 0)),
                   pl.BlockSpec((BLK, wr_sh[0]), lambda i: (i, 0))],
        out_shape=[jax.ShapeDtypeStruct((N, wl_sh[0]), dtypes[0]),
                   jax.ShapeDtypeStruct((N, wr_sh[0]), dtypes[1])],
    )(Sp, Cp, R, b, g, be, rm, rv, Wln, Wrn)


def _tc_fin(Sp, Cp, R, b, C_out):
    def body(s_ref, c_ref, r_ref, b_ref, o_ref):
        ssum = s_ref[0] + s_ref[1]
        cnt = c_ref[0, :, 0:1] + c_ref[1, :, 0:1]
        o = ssum / jnp.maximum(cnt, 1.0) + b_ref[...] + r_ref[...]
        o_ref[...] = o[:, :C_out]

    return pl.pallas_call(
        body,
        grid=(N // BLK,),
        in_specs=[pl.BlockSpec((NC, BLK, CW), lambda i: (0, i, 0)),
                  pl.BlockSpec((NC, BLK, CW), lambda i: (0, i, 0)),
                  pl.BlockSpec((BLK, CW), lambda i: (i, 0)),
                  pl.BlockSpec((CW,), lambda i: (0,))],
        out_specs=pl.BlockSpec((BLK, C_out), lambda i: (i, 0)),
        out_shape=jax.ShapeDtypeStruct((N, C_out), jnp.float32),
    )(Sp, Cp, R, b)


def kernel(x, edge_index, Wl0, Wr0, b0, g0, be0, rm0, rv0,
           Wl1, Wr1, b1, g1, be1, rm1, rv1, Wl2, Wr2, b2):
    C_out = Wl2.shape[0]

    pad = ((0, CW - C_out), (0, 0))
    Wl2p = jnp.pad(Wl2, pad)
    Wr2p = jnp.pad(Wr2, pad)
    b2p = jnp.pad(b2, (0, CW - C_out))

    npad_edges = NW * EPW - E
    iota_pad = jnp.arange(npad_edges, dtype=jnp.int32)
    src_p = jnp.concatenate(
        [edge_index[0], jax.lax.rem(iota_pad * 37, jnp.int32(N))])
    dst_p = jnp.concatenate(
        [edge_index[1],
         N + jax.lax.rem(iota_pad * 37, jnp.int32(NPAD - N))])
    src3 = src_p.reshape(NW, RND, K)
    dst3 = dst_p.reshape(NW, RND, K)
    ones16 = jnp.ones((K, CW), jnp.float32)
    zbf = jnp.zeros((SPT, D), jnp.bfloat16)
    z16 = jnp.zeros((SPT, CW), jnp.float32)
    f32, bf16 = jnp.float32, jnp.bfloat16

    P0, R0 = _tc_matmul(x, Wl0, Wr0, (bf16, f32))
    S0, CNT = jax.tree.leaves(
        _segsum(1, D, bf16, True)(P0, src3, dst3, ones16, zbf, z16))
    P1, R1 = _tc_mid(S0, CNT, R0, b0, g0, be0, rm0, rv0,
                     Wl1, Wr1, (bf16, f32))
    (S1,) = jax.tree.leaves(
        _segsum(1, D, bf16, False)(P1, src3, dst3, ones16, zbf, z16))
    P2, R2 = _tc_mid(S1, CNT, R1, b1, g1, be1, rm1, rv1,
                     Wl2p, Wr2p, (f32, f32))
    (S2,) = jax.tree.leaves(
        _segsum(1, CW, f32, False)(P2, src3, dst3, ones16, z16, z16))
    return _tc_fin(S2, CNT, R2, b2p, C_out)
